# baseline (speedup 1.0000x reference)
import functools

import jax
import jax.numpy as jnp
from jax import lax
from jax.experimental import pallas as pl
from jax.experimental.pallas import tpu as pltpu
from jax.experimental.pallas import tpu_sc as plsc

N = 10000
D = 256
E = 160000
NPAD = 10112
EPAD = 163840
B = 64
EPT = EPAD // 16
NB1 = EPT // B
EPW = EPAD // 32
NB2 = EPW // B
RPT = NPAD // 16
K1 = 32
K2 = 16
HB = D // 2

_mesh = plsc.VectorSubcoreMesh(core_axis_name="c", subcore_axis_name="s")


@functools.partial(
    pl.kernel,
    mesh=_mesh,
    out_type=(
        jax.ShapeDtypeStruct((NPAD, HB), jnp.float32),
        jax.ShapeDtypeStruct((NPAD, HB), jnp.float32),
        jax.ShapeDtypeStruct((NPAD, HB), jnp.float32),
        jax.ShapeDtypeStruct((NPAD, HB), jnp.float32),
    ),
    scratch_types=(
        pltpu.VMEM((K1, B), jnp.int32),
        pltpu.VMEM((K1, B), jnp.int32),
        pltpu.VMEM((K2, B), jnp.int32),
        pltpu.VMEM((B, HB), jnp.float32),
        pltpu.VMEM((B, HB), jnp.float32),
        pltpu.VMEM_SHARED((NPAD, HB), jnp.float32),
        pltpu.SemaphoreType.DMA,
        pltpu.SemaphoreType.DMA,
        pltpu.SemaphoreType.DMA,
        pltpu.SemaphoreType.DMA,
    ),
)
def _sc_aggregate(f0_hbm, f1_hbm, src2_hbm, dst2_hbm,
                  acc0_out, acc1_out, deg0_out, deg1_out,
                  src_i, dst_i, dst2_i, rows_a, rows_b,
                  acc_sh, sem_a, sem_b, sem_sa, sem_sb):
    c = lax.axis_index("c")
    s = lax.axis_index("s")
    row0 = s * RPT
    zeros = jnp.zeros((16,), jnp.float32)
    ones = jnp.ones((16,), jnp.float32)
    nfull = RPT // B
    tail = RPT - nfull * B

    def _fill(buf, vec):
        def body(i, carry):
            for j in range(HB // 16):
                buf[i, pl.ds(j * 16, 16)] = vec
            return carry
        lax.fori_loop(0, B, body, 0)

    def _zero_acc():
        for k in range(nfull):
            pltpu.sync_copy(rows_b, acc_sh.at[pl.ds(row0 + k * B, B)])
        pltpu.sync_copy(rows_b.at[pl.ds(0, tail)],
                        acc_sh.at[pl.ds(row0 + nfull * B, tail)])

    def _copy_out(hbm_ref):
        for k in range(nfull):
            r = row0 + k * B
            pltpu.sync_copy(acc_sh.at[pl.ds(r, B)], rows_a)
            pltpu.sync_copy(rows_a, hbm_ref.at[pl.ds(r, B)])
        r = row0 + nfull * B
        pltpu.sync_copy(acc_sh.at[pl.ds(r, tail)], rows_a.at[pl.ds(0, tail)])
        pltpu.sync_copy(rows_a.at[pl.ds(0, tail)], hbm_ref.at[pl.ds(r, tail)])

    _fill(rows_b, zeros)
    _zero_acc()
    plsc.subcore_barrier()

    def _edge_phase(tbl_ref):
        def chunk(nc, carry):
            pltpu.sync_copy(src2_hbm.at[pl.ds(s * NB1 + nc * K1, K1)], src_i)
            pltpu.sync_copy(dst2_hbm.at[pl.ds(s * NB1 + nc * K1, K1)], dst_i)
            pltpu.async_copy(tbl_ref.at[src_i.at[0]], rows_a, sem_a)

            def body(g, carry2):
                i = 2 * g
                pltpu.async_copy(tbl_ref.at[src_i.at[i + 1]], rows_b, sem_b)
                pltpu.make_async_copy(tbl_ref.at[src_i.at[i]], rows_a, sem_a).wait()
                pltpu.async_copy(rows_a, acc_sh.at[dst_i.at[i]], sem_sa, add=True)
                pltpu.make_async_copy(tbl_ref.at[src_i.at[i + 1]], rows_b, sem_b).wait()
                pltpu.async_copy(rows_b, acc_sh.at[dst_i.at[i + 1]], sem_sb, add=True)
                pltpu.make_async_copy(rows_a, acc_sh.at[dst_i.at[i]], sem_sa).wait()

                @pl.when(g < K1 // 2 - 1)
                def _():
                    pltpu.async_copy(tbl_ref.at[src_i.at[i + 2]], rows_a, sem_a)

                pltpu.make_async_copy(rows_b, acc_sh.at[dst_i.at[i + 1]], sem_sb).wait()
                return carry2
            lax.fori_loop(0, K1 // 2, body, 0)
            return carry
        lax.fori_loop(0, NB1 // K1, chunk, 0)

    @pl.when(c == 0)
    def _():
        _edge_phase(f0_hbm)

    @pl.when(c == 1)
    def _():
        _edge_phase(f1_hbm)

    plsc.subcore_barrier()

    @pl.when(c == 0)
    def _():
        _copy_out(acc0_out)

    @pl.when(c == 1)
    def _():
        _copy_out(acc1_out)

    plsc.subcore_barrier()

    _fill(rows_b, zeros)
    _zero_acc()
    _fill(rows_b, ones)
    plsc.subcore_barrier()

    wbase = (c * 16 + s) * NB2

    def _deg_chunk(nc, carry):
        pltpu.sync_copy(dst2_hbm.at[pl.ds(wbase + nc * K2, K2)], dst2_i)

        def fire(i, carry2):
            pltpu.async_copy(rows_b, acc_sh.at[dst2_i.at[i]], sem_b, add=True)
            return carry2
        lax.fori_loop(0, K2, fire, 0)

        def drain(i, carry2):
            pltpu.make_async_copy(rows_b, acc_sh.at[dst2_i.at[i]], sem_b).wait()
            return carry2
        lax.fori_loop(0, K2, drain, 0)
        return carry
    lax.fori_loop(0, NB2 // K2, _deg_chunk, 0)

    plsc.subcore_barrier()

    @pl.when(c == 0)
    def _():
        _copy_out(deg0_out)

    @pl.when(c == 1)
    def _():
        _copy_out(deg1_out)


RB = 1000


def _tc_body(feat_ref, a0_ref, a1_ref, d0_ref, d1_ref, wn_ref, ws_ref, b_ref,
             out_ref):
    x = feat_ref[...]
    d = d0_ref[...][:, 0:1] + d1_ref[...][:, 0:1]
    neigh = jnp.concatenate([a0_ref[...], a1_ref[...]], axis=1)
    agg = jnp.where(d > 0.0, neigh / jnp.maximum(d, 1.0), 0.0)
    hs = lax.dot_general(x, ws_ref[...], (((1,), (1,)), ((), ())),
                         preferred_element_type=jnp.float32)
    hn = lax.dot_general(agg, wn_ref[...], (((1,), (1,)), ((), ())),
                         preferred_element_type=jnp.float32)
    out_ref[...] = hs + hn + b_ref[...]


_tc_combine = pl.pallas_call(
    _tc_body,
    grid=(N // RB,),
    in_specs=[
        pl.BlockSpec((RB, D), lambda i: (i, 0)),
        pl.BlockSpec((RB, HB), lambda i: (i, 0)),
        pl.BlockSpec((RB, HB), lambda i: (i, 0)),
        pl.BlockSpec((RB, HB), lambda i: (i, 0)),
        pl.BlockSpec((RB, HB), lambda i: (i, 0)),
        pl.BlockSpec((D, D), lambda i: (0, 0)),
        pl.BlockSpec((D, D), lambda i: (0, 0)),
        pl.BlockSpec((1, D), lambda i: (0, 0)),
    ],
    out_specs=pl.BlockSpec((RB, D), lambda i: (i, 0)),
    out_shape=jax.ShapeDtypeStruct((N, D), jnp.float32),
)


@jax.jit
def kernel(feat, edge_index, W_neigh, W_self, b_self):
    src = edge_index[0].astype(jnp.int32)
    dst = edge_index[1].astype(jnp.int32)
    pad = EPAD - E
    src = jnp.concatenate([src, jnp.zeros((pad,), jnp.int32)])
    dst = jnp.concatenate([dst, jnp.full((pad,), N, jnp.int32)])
    src2 = src.reshape(EPAD // B, B)
    dst2 = dst.reshape(EPAD // B, B)
    f0 = feat[:, :HB]
    f1 = feat[:, HB:]
    acc0, acc1, deg0, deg1 = _sc_aggregate(f0, f1, src2, dst2)
    return _tc_combine(feat, acc0, acc1, deg0, deg1, W_neigh, W_self,
                       b_self.reshape(1, D))


# speedup vs baseline: 1.0921x; 1.0921x over previous
import functools

import jax
import jax.numpy as jnp
from jax import lax
from jax.experimental import pallas as pl
from jax.experimental.pallas import tpu as pltpu
from jax.experimental.pallas import tpu_sc as plsc

N = 10000
D = 256
E = 160000
NPAD = 10112
EPAD = 163840
B = 64
EPT = EPAD // 16
NB1 = EPT // B
EPW = EPAD // 32
NB2 = EPW // B
RPT = NPAD // 16
K1 = 40
K2 = 40
HB = D // 2

_mesh = plsc.VectorSubcoreMesh(core_axis_name="c", subcore_axis_name="s")


@functools.partial(
    pl.kernel,
    mesh=_mesh,
    out_type=(
        jax.ShapeDtypeStruct((NPAD, HB), jnp.float32),
        jax.ShapeDtypeStruct((NPAD, HB), jnp.float32),
        jax.ShapeDtypeStruct((NPAD, HB), jnp.float32),
        jax.ShapeDtypeStruct((NPAD, HB), jnp.float32),
    ),
    scratch_types=(
        pltpu.VMEM((K1, B), jnp.int32),
        pltpu.VMEM((K1, B), jnp.int32),
        pltpu.VMEM((K2, B), jnp.int32),
        pltpu.VMEM((B, HB), jnp.float32),
        pltpu.VMEM((B, HB), jnp.float32),
        pltpu.VMEM_SHARED((NPAD, HB), jnp.float32),
        pltpu.SemaphoreType.DMA,
        pltpu.SemaphoreType.DMA,
    ),
)
def _sc_aggregate(f0_hbm, f1_hbm, src2_hbm, dst2_hbm,
                  acc0_out, acc1_out, deg0_out, deg1_out,
                  src_i, dst_i, dst2_i, rows_a, rows_b,
                  acc_sh, sem_a, sem_b):
    c = lax.axis_index("c")
    s = lax.axis_index("s")
    row0 = s * RPT
    zeros = jnp.zeros((16,), jnp.float32)
    ones = jnp.ones((16,), jnp.float32)
    nfull = RPT // B
    tail = RPT - nfull * B

    def _fill(buf, vec):
        def body(i, carry):
            for j in range(HB // 16):
                buf[i, pl.ds(j * 16, 16)] = vec
            return carry
        lax.fori_loop(0, B, body, 0)

    def _zero_acc():
        for k in range(nfull):
            pltpu.sync_copy(rows_b, acc_sh.at[pl.ds(row0 + k * B, B)])
        pltpu.sync_copy(rows_b.at[pl.ds(0, tail)],
                        acc_sh.at[pl.ds(row0 + nfull * B, tail)])

    def _copy_out(hbm_ref):
        for k in range(nfull):
            r = row0 + k * B
            pltpu.sync_copy(acc_sh.at[pl.ds(r, B)], rows_a)
            pltpu.sync_copy(rows_a, hbm_ref.at[pl.ds(r, B)])
        r = row0 + nfull * B
        pltpu.sync_copy(acc_sh.at[pl.ds(r, tail)], rows_a.at[pl.ds(0, tail)])
        pltpu.sync_copy(rows_a.at[pl.ds(0, tail)], hbm_ref.at[pl.ds(r, tail)])

    _fill(rows_b, zeros)
    _zero_acc()
    plsc.subcore_barrier()

    def _edge_phase(tbl_ref):
        def chunk(nc, carry):
            pltpu.sync_copy(src2_hbm.at[pl.ds(s * NB1 + nc * K1, K1)], src_i)
            pltpu.sync_copy(dst2_hbm.at[pl.ds(s * NB1 + nc * K1, K1)], dst_i)
            pltpu.async_copy(tbl_ref.at[src_i.at[0]], rows_a, sem_a)

            def body(g, carry2):
                i = 2 * g
                pltpu.async_copy(tbl_ref.at[src_i.at[i + 1]], rows_b, sem_b)
                pltpu.make_async_copy(tbl_ref.at[src_i.at[i]], rows_a, sem_a).wait()
                pltpu.sync_copy(rows_a, acc_sh.at[dst_i.at[i]], add=True)

                @pl.when(g < K1 // 2 - 1)
                def _():
                    pltpu.async_copy(tbl_ref.at[src_i.at[i + 2]], rows_a, sem_a)

                pltpu.make_async_copy(tbl_ref.at[src_i.at[i + 1]], rows_b, sem_b).wait()
                pltpu.sync_copy(rows_b, acc_sh.at[dst_i.at[i + 1]], add=True)
                return carry2
            lax.fori_loop(0, K1 // 2, body, 0)
            return carry
        lax.fori_loop(0, NB1 // K1, chunk, 0)

    @pl.when(c == 0)
    def _():
        _edge_phase(f0_hbm)

    @pl.when(c == 1)
    def _():
        _edge_phase(f1_hbm)

    plsc.subcore_barrier()

    @pl.when(c == 0)
    def _():
        _copy_out(acc0_out)

    @pl.when(c == 1)
    def _():
        _copy_out(acc1_out)

    plsc.subcore_barrier()

    _fill(rows_b, zeros)
    _zero_acc()
    _fill(rows_b, ones)
    plsc.subcore_barrier()

    wbase = (c * 16 + s) * NB2

    def _deg_chunk(nc, carry):
        pltpu.sync_copy(dst2_hbm.at[pl.ds(wbase + nc * K2, K2)], dst2_i)

        def fire(i, carry2):
            pltpu.async_copy(rows_b, acc_sh.at[dst2_i.at[i]], sem_b, add=True)
            return carry2
        lax.fori_loop(0, K2, fire, 0)

        def drain(i, carry2):
            pltpu.make_async_copy(rows_b, acc_sh.at[dst2_i.at[i]], sem_b).wait()
            return carry2
        lax.fori_loop(0, K2, drain, 0)
        return carry
    lax.fori_loop(0, NB2 // K2, _deg_chunk, 0)

    plsc.subcore_barrier()

    @pl.when(c == 0)
    def _():
        _copy_out(deg0_out)

    @pl.when(c == 1)
    def _():
        _copy_out(deg1_out)


RB = 1000


def _tc_body(feat_ref, a0_ref, a1_ref, d0_ref, d1_ref, wn_ref, ws_ref, b_ref,
             out_ref):
    x = feat_ref[...]
    d = d0_ref[...][:, 0:1] + d1_ref[...][:, 0:1]
    neigh = jnp.concatenate([a0_ref[...], a1_ref[...]], axis=1)
    agg = jnp.where(d > 0.0, neigh / jnp.maximum(d, 1.0), 0.0)
    hs = lax.dot_general(x, ws_ref[...], (((1,), (1,)), ((), ())),
                         preferred_element_type=jnp.float32)
    hn = lax.dot_general(agg, wn_ref[...], (((1,), (1,)), ((), ())),
                         preferred_element_type=jnp.float32)
    out_ref[...] = hs + hn + b_ref[...]


_tc_combine = pl.pallas_call(
    _tc_body,
    grid=(N // RB,),
    in_specs=[
        pl.BlockSpec((RB, D), lambda i: (i, 0)),
        pl.BlockSpec((RB, HB), lambda i: (i, 0)),
        pl.BlockSpec((RB, HB), lambda i: (i, 0)),
        pl.BlockSpec((RB, HB), lambda i: (i, 0)),
        pl.BlockSpec((RB, HB), lambda i: (i, 0)),
        pl.BlockSpec((D, D), lambda i: (0, 0)),
        pl.BlockSpec((D, D), lambda i: (0, 0)),
        pl.BlockSpec((1, D), lambda i: (0, 0)),
    ],
    out_specs=pl.BlockSpec((RB, D), lambda i: (i, 0)),
    out_shape=jax.ShapeDtypeStruct((N, D), jnp.float32),
)


@jax.jit
def kernel(feat, edge_index, W_neigh, W_self, b_self):
    src = edge_index[0].astype(jnp.int32)
    dst = edge_index[1].astype(jnp.int32)
    pad = EPAD - E
    src = jnp.concatenate([src, jnp.zeros((pad,), jnp.int32)])
    dst = jnp.concatenate([dst, jnp.full((pad,), N, jnp.int32)])
    src2 = src.reshape(EPAD // B, B)
    dst2 = dst.reshape(EPAD // B, B)
    f0 = feat[:, :HB]
    f1 = feat[:, HB:]
    acc0, acc1, deg0, deg1 = _sc_aggregate(f0, f1, src2, dst2)
    return _tc_combine(feat, acc0, acc1, deg0, deg1, W_neigh, W_self,
                       b_self.reshape(1, D))
